# baseline (speedup 1.0000x reference)
import jax
import jax.numpy as jnp
from jax import lax
from jax.experimental import pallas as pl
from jax.experimental.pallas import tpu as pltpu
from jax.experimental.pallas import tpu_sc as plsc

BATCH = 16384
EMB = 128
L = 16
NC, NS = 2, 16
NW = NC * NS
R = BATCH // NW
C = 128
NCHUNK = R // C
GROUPS = C // L

_GATHER_DN = lax.GatherDimensionNumbers(
    offset_dims=(), collapsed_slice_dims=(0,), start_index_map=(0,))


def _shuffle(v, idx):
    return lax.gather(v, idx[:, None], _GATHER_DN, (1,),
                      mode=lax.GatherScatterMode.PROMISE_IN_BOUNDS)


def _body(uidx_hbm, iidx_hbm, uemb_hbm, iemb_hbm, ub_hbm, ib_hbm, out_hbm,
          uidx_v, iidx_v, urow_v, irow_v, ubv, ibv, out_v, sems):
    cid = lax.axis_index("c")
    sid = lax.axis_index("s")
    wid = sid * NC + cid
    base = wid * R
    lanes = lax.iota(jnp.int32, L)
    zero = jnp.zeros((L,), jnp.float32)

    pltpu.sync_copy(uidx_hbm.at[pl.ds(base, R)], uidx_v)
    pltpu.sync_copy(iidx_hbm.at[pl.ds(base, R)], iidx_v)

    def start_chunk(ci, buf):
        uix = uidx_v.at[pl.ds(ci * C, C)]
        iix = iidx_v.at[pl.ds(ci * C, C)]
        pltpu.async_copy(uemb_hbm.at[uix], urow_v.at[buf], sems.at[buf, 0])
        pltpu.async_copy(iemb_hbm.at[iix], irow_v.at[buf], sems.at[buf, 1])
        pltpu.async_copy(ub_hbm.at[uix], ubv.at[buf], sems.at[buf, 2])
        pltpu.async_copy(ib_hbm.at[iix], ibv.at[buf], sems.at[buf, 3])

    def wait_chunk(buf):
        pltpu.make_async_copy(uemb_hbm.at[pl.ds(0, C)], urow_v.at[buf],
                              sems.at[buf, 0]).wait()
        pltpu.make_async_copy(iemb_hbm.at[pl.ds(0, C)], irow_v.at[buf],
                              sems.at[buf, 1]).wait()
        pltpu.make_async_copy(ub_hbm.at[pl.ds(0, C)], ubv.at[buf],
                              sems.at[buf, 2]).wait()
        pltpu.make_async_copy(ib_hbm.at[pl.ds(0, C)], ibv.at[buf],
                              sems.at[buf, 3]).wait()

    start_chunk(0, 0)
    for ci in range(NCHUNK):
        buf = ci % 2
        if ci + 1 < NCHUNK:
            start_chunk(ci + 1, 1 - buf)
        wait_chunk(buf)

        def group_body(g, carry2):
            row0 = g * L
            acc = jnp.zeros((L,), jnp.float32)
            for r in range(L):
                row = row0 + r
                t = [urow_v[buf, row, pl.ds(j * L, L)] *
                     irow_v[buf, row, pl.ds(j * L, L)]
                     for j in range(EMB // L)]
                while len(t) > 1:
                    t = [t[2 * i] + t[2 * i + 1] for i in range(len(t) // 2)]
                s = t[0]
                for sh in (8, 4, 2, 1):
                    s = s + _shuffle(s, lanes ^ sh)
                acc = jnp.where(lanes == r, s, acc)
            acc = acc + ubv[buf, pl.ds(row0, L)] + ibv[buf, pl.ds(row0, L)]
            out_v[pl.ds(ci * C + row0, L)] = jnp.maximum(acc, zero)
            return carry2

        lax.fori_loop(0, GROUPS, group_body, 0, unroll=False)

    pltpu.sync_copy(out_v, out_hbm.at[pl.ds(base, R)])


@jax.jit
def kernel(user_idx, item_idx, user_emb, item_emb, user_bias, item_bias):
    mesh = plsc.VectorSubcoreMesh(core_axis_name="c", subcore_axis_name="s")
    run = pl.kernel(
        _body,
        out_type=jax.ShapeDtypeStruct((BATCH,), jnp.float32),
        mesh=mesh,
        scratch_types=[
            pltpu.VMEM((R,), jnp.int32),
            pltpu.VMEM((R,), jnp.int32),
            pltpu.VMEM((2, C, EMB), jnp.float32),
            pltpu.VMEM((2, C, EMB), jnp.float32),
            pltpu.VMEM((2, C), jnp.float32),
            pltpu.VMEM((2, C), jnp.float32),
            pltpu.VMEM((R,), jnp.float32),
            pltpu.SemaphoreType.DMA((2, 4)),
        ],
    )
    return run(user_idx.astype(jnp.int32), item_idx.astype(jnp.int32),
               user_emb, item_emb,
               user_bias.reshape(-1), item_bias.reshape(-1))


# speedup vs baseline: 1.4028x; 1.4028x over previous
import jax
import jax.numpy as jnp
from jax import lax
from jax.experimental import pallas as pl
from jax.experimental.pallas import tpu as pltpu
from jax.experimental.pallas import tpu_sc as plsc

BATCH = 16384
EMB = 128
L = 16
NC, NS = 2, 16
NW = NC * NS
R = BATCH // NW
C = 128
NCHUNK = R // C
GROUPS = C // L

_GATHER_DN = lax.GatherDimensionNumbers(
    offset_dims=(), collapsed_slice_dims=(0,), start_index_map=(0,))


def _shuffle(v, idx):
    return lax.gather(v, idx[:, None], _GATHER_DN, (1,),
                      mode=lax.GatherScatterMode.PROMISE_IN_BOUNDS)


def _body(uidx_hbm, iidx_hbm, uemb_hbm, iemb_hbm, ub_hbm, ib_hbm, out_hbm,
          uidx_v, iidx_v, urow_v, irow_v, ubv, ibv, out_v, sems):
    cid = lax.axis_index("c")
    sid = lax.axis_index("s")
    wid = sid * NC + cid
    base = wid * R
    lanes = lax.iota(jnp.int32, L)
    zero = jnp.zeros((L,), jnp.float32)
    masks = [(lanes & sh) == 0 for sh in (8, 4, 2, 1)]
    xors = [lanes ^ sh for sh in (8, 4, 2, 1)]

    pltpu.sync_copy(uidx_hbm.at[pl.ds(base, R)], uidx_v)
    pltpu.sync_copy(iidx_hbm.at[pl.ds(base, R)], iidx_v)

    def start_chunk(ci, buf):
        uix = uidx_v.at[pl.ds(ci * C, C)]
        iix = iidx_v.at[pl.ds(ci * C, C)]
        pltpu.async_copy(uemb_hbm.at[uix], urow_v.at[buf], sems.at[buf, 0])
        pltpu.async_copy(iemb_hbm.at[iix], irow_v.at[buf], sems.at[buf, 1])
        pltpu.async_copy(ub_hbm.at[uix], ubv.at[buf], sems.at[buf, 2])
        pltpu.async_copy(ib_hbm.at[iix], ibv.at[buf], sems.at[buf, 3])

    def wait_chunk(buf):
        pltpu.make_async_copy(uemb_hbm.at[pl.ds(0, C)], urow_v.at[buf],
                              sems.at[buf, 0]).wait()
        pltpu.make_async_copy(iemb_hbm.at[pl.ds(0, C)], irow_v.at[buf],
                              sems.at[buf, 1]).wait()
        pltpu.make_async_copy(ub_hbm.at[pl.ds(0, C)], ubv.at[buf],
                              sems.at[buf, 2]).wait()
        pltpu.make_async_copy(ib_hbm.at[pl.ds(0, C)], ibv.at[buf],
                              sems.at[buf, 3]).wait()

    def combine(a, b, d):
        return (jnp.where(masks[d], a, b) +
                _shuffle(jnp.where(masks[d], b, a), xors[d]))

    start_chunk(0, 0)
    for ci in range(NCHUNK):
        buf = ci % 2
        if ci + 1 < NCHUNK:
            start_chunk(ci + 1, 1 - buf)
        wait_chunk(buf)

        def group_body(g, carry2):
            row0 = g * L
            vs = []
            for r in (0, 8, 4, 12, 2, 10, 6, 14, 1, 9, 5, 13, 3, 11, 7, 15):
                row = row0 + r
                s = (urow_v[buf, row, pl.ds(0, L)] *
                     irow_v[buf, row, pl.ds(0, L)])
                for j in range(1, EMB // L):
                    s = s + (urow_v[buf, row, pl.ds(j * L, L)] *
                             irow_v[buf, row, pl.ds(j * L, L)])
                vs.append(s)
            for d in range(4):
                vs = [combine(vs[2 * i], vs[2 * i + 1], d)
                      for i in range(len(vs) // 2)]
            acc = vs[0] + ubv[buf, pl.ds(row0, L)] + ibv[buf, pl.ds(row0, L)]
            out_v[pl.ds(ci * C + row0, L)] = jnp.maximum(acc, zero)
            return carry2

        lax.fori_loop(0, GROUPS, group_body, 0, unroll=False)

    pltpu.sync_copy(out_v, out_hbm.at[pl.ds(base, R)])


@jax.jit
def kernel(user_idx, item_idx, user_emb, item_emb, user_bias, item_bias):
    mesh = plsc.VectorSubcoreMesh(core_axis_name="c", subcore_axis_name="s")
    run = pl.kernel(
        _body,
        out_type=jax.ShapeDtypeStruct((BATCH,), jnp.float32),
        mesh=mesh,
        scratch_types=[
            pltpu.VMEM((R,), jnp.int32),
            pltpu.VMEM((R,), jnp.int32),
            pltpu.VMEM((2, C, EMB), jnp.float32),
            pltpu.VMEM((2, C, EMB), jnp.float32),
            pltpu.VMEM((2, C), jnp.float32),
            pltpu.VMEM((2, C), jnp.float32),
            pltpu.VMEM((R,), jnp.float32),
            pltpu.SemaphoreType.DMA((2, 4)),
        ],
    )
    return run(user_idx.astype(jnp.int32), item_idx.astype(jnp.int32),
               user_emb, item_emb,
               user_bias.reshape(-1), item_bias.reshape(-1))


# speedup vs baseline: 1.4048x; 1.0014x over previous
import jax
import jax.numpy as jnp
from jax import lax
from jax.experimental import pallas as pl
from jax.experimental.pallas import tpu as pltpu
from jax.experimental.pallas import tpu_sc as plsc

BATCH = 16384
EMB = 128
L = 16
NC, NS = 2, 16
NW = NC * NS
R = BATCH // NW
C = 128
NCHUNK = R // C
GROUPS = C // L

_GATHER_DN = lax.GatherDimensionNumbers(
    offset_dims=(), collapsed_slice_dims=(0,), start_index_map=(0,))


def _shuffle(v, idx):
    return lax.gather(v, idx[:, None], _GATHER_DN, (1,),
                      mode=lax.GatherScatterMode.PROMISE_IN_BOUNDS)


def _body(uidx_hbm, iidx_hbm, uemb_hbm, iemb_hbm, ub_hbm, ib_hbm, out_hbm,
          uidx_v, iidx_v, urow_v, irow_v, ubv, ibv, out_v, sems):
    cid = lax.axis_index("c")
    sid = lax.axis_index("s")
    wid = sid * NC + cid
    base = wid * R
    lanes = lax.iota(jnp.int32, L)
    zero = jnp.zeros((L,), jnp.float32)
    masks = [(lanes & sh) == 0 for sh in (8, 4, 2, 1)]
    xors = [lanes ^ sh for sh in (8, 4, 2, 1)]

    pltpu.sync_copy(uidx_hbm.at[pl.ds(base, R)], uidx_v)
    pltpu.sync_copy(iidx_hbm.at[pl.ds(base, R)], iidx_v)

    def start_chunk(ci, buf):
        uix = uidx_v.at[pl.ds(ci * C, C)]
        iix = iidx_v.at[pl.ds(ci * C, C)]
        pltpu.async_copy(uemb_hbm.at[uix], urow_v.at[buf], sems.at[buf, 0])
        pltpu.async_copy(iemb_hbm.at[iix], irow_v.at[buf], sems.at[buf, 1])
        pltpu.async_copy(ub_hbm.at[uix], ubv.at[buf], sems.at[buf, 2])
        pltpu.async_copy(ib_hbm.at[iix], ibv.at[buf], sems.at[buf, 3])

    def wait_chunk(buf):
        pltpu.make_async_copy(uemb_hbm.at[pl.ds(0, C)], urow_v.at[buf],
                              sems.at[buf, 0]).wait()
        pltpu.make_async_copy(iemb_hbm.at[pl.ds(0, C)], irow_v.at[buf],
                              sems.at[buf, 1]).wait()
        pltpu.make_async_copy(ub_hbm.at[pl.ds(0, C)], ubv.at[buf],
                              sems.at[buf, 2]).wait()
        pltpu.make_async_copy(ib_hbm.at[pl.ds(0, C)], ibv.at[buf],
                              sems.at[buf, 3]).wait()

    def combine(a, b, d):
        return (jnp.where(masks[d], a, b) +
                _shuffle(jnp.where(masks[d], b, a), xors[d]))

    start_chunk(0, 0)
    for ci in range(NCHUNK):
        buf = ci % 2
        if ci + 1 < NCHUNK:
            start_chunk(ci + 1, 1 - buf)
        wait_chunk(buf)

        def group_body(g, carry2):
            row0 = g * L
            vs = []
            for r in (0, 8, 4, 12, 2, 10, 6, 14, 1, 9, 5, 13, 3, 11, 7, 15):
                row = row0 + r
                s = (urow_v[buf, row, pl.ds(0, L)] *
                     irow_v[buf, row, pl.ds(0, L)])
                for j in range(1, EMB // L):
                    s = s + (urow_v[buf, row, pl.ds(j * L, L)] *
                             irow_v[buf, row, pl.ds(j * L, L)])
                vs.append(s)
            for d in range(4):
                vs = [combine(vs[2 * i], vs[2 * i + 1], d)
                      for i in range(len(vs) // 2)]
            acc = vs[0] + ubv[buf, pl.ds(row0, L)] + ibv[buf, pl.ds(row0, L)]
            out_v[pl.ds(ci * C + row0, L)] = jnp.maximum(acc, zero)
            return carry2

        lax.fori_loop(0, GROUPS, group_body, 0, unroll=False)

    pltpu.sync_copy(out_v, out_hbm.at[pl.ds(base, R)])


@jax.jit
def kernel(user_idx, item_idx, user_emb, item_emb, user_bias, item_bias):
    mesh = plsc.VectorSubcoreMesh(core_axis_name="c", subcore_axis_name="s")
    run = pl.kernel(
        _body,
        out_type=jax.ShapeDtypeStruct((BATCH,), jnp.float32),
        mesh=mesh,
        scratch_types=[
            pltpu.VMEM((R,), jnp.int32),
            pltpu.VMEM((R,), jnp.int32),
            pltpu.VMEM((2, C, EMB), jnp.float32),
            pltpu.VMEM((2, C, EMB), jnp.float32),
            pltpu.VMEM((2, C), jnp.float32),
            pltpu.VMEM((2, C), jnp.float32),
            pltpu.VMEM((R,), jnp.float32),
            pltpu.SemaphoreType.DMA((2, 4)),
        ],
    )
    return run(user_idx.astype(jnp.int32), item_idx.astype(jnp.int32),
               user_emb, item_emb,
               user_bias[:, 0], item_bias[:, 0])


# speedup vs baseline: 1.4232x; 1.0131x over previous
import jax
import jax.numpy as jnp
from jax import lax
from jax.experimental import pallas as pl
from jax.experimental.pallas import tpu as pltpu
from jax.experimental.pallas import tpu_sc as plsc

BATCH = 16384
EMB = 128
L = 16
NC, NS = 2, 16
NW = NC * NS
R = BATCH // NW
C = 128
NCHUNK = R // C
GROUPS = C // L

_GATHER_DN = lax.GatherDimensionNumbers(
    offset_dims=(), collapsed_slice_dims=(0,), start_index_map=(0,))


def _shuffle(v, idx):
    return lax.gather(v, idx[:, None], _GATHER_DN, (1,),
                      mode=lax.GatherScatterMode.PROMISE_IN_BOUNDS)


def _body(uidx_hbm, iidx_hbm, uemb_hbm, iemb_hbm, ub_hbm, ib_hbm, out_hbm,
          uidx_v, iidx_v, urow_v, irow_v, ubv, ibv, out_v, sems):
    cid = lax.axis_index("c")
    sid = lax.axis_index("s")
    wid = sid * NC + cid
    base = wid * R
    lanes = lax.iota(jnp.int32, L)
    zero = jnp.zeros((L,), jnp.float32)
    masks = [(lanes & sh) == 0 for sh in (8, 4, 2, 1)]
    xors = [lanes ^ sh for sh in (8, 4, 2, 1)]

    pltpu.sync_copy(uidx_hbm.at[pl.ds(base, R)], uidx_v)
    pltpu.sync_copy(iidx_hbm.at[pl.ds(base, R)], iidx_v)

    def start_chunk(ci, buf):
        uix = uidx_v.at[pl.ds(ci * C, C)]
        iix = iidx_v.at[pl.ds(ci * C, C)]
        pltpu.async_copy(uemb_hbm.at[uix], urow_v.at[buf], sems.at[buf, 0])
        pltpu.async_copy(iemb_hbm.at[iix], irow_v.at[buf], sems.at[buf, 1])
        pltpu.async_copy(ub_hbm.at[uix], ubv.at[buf], sems.at[buf, 2])
        pltpu.async_copy(ib_hbm.at[iix], ibv.at[buf], sems.at[buf, 3])

    def wait_chunk(buf):
        pltpu.make_async_copy(uemb_hbm.at[pl.ds(0, C)], urow_v.at[buf],
                              sems.at[buf, 0]).wait()
        pltpu.make_async_copy(iemb_hbm.at[pl.ds(0, C)], irow_v.at[buf],
                              sems.at[buf, 1]).wait()
        pltpu.make_async_copy(ub_hbm.at[pl.ds(0, C)], ubv.at[buf],
                              sems.at[buf, 2]).wait()
        pltpu.make_async_copy(ib_hbm.at[pl.ds(0, C)], ibv.at[buf],
                              sems.at[buf, 3]).wait()

    def combine(a, b, d):
        return (jnp.where(masks[d], a, b) +
                _shuffle(jnp.where(masks[d], b, a), xors[d]))

    start_chunk(0, 0)

    def chunk_body(ci, carry):
        buf = lax.rem(ci, 2)

        @pl.when(ci + 1 < NCHUNK)
        def _():
            start_chunk(ci + 1, 1 - buf)

        wait_chunk(buf)

        def group_body(g, carry2):
            row0 = g * L
            vs = []
            for r in (0, 8, 4, 12, 2, 10, 6, 14, 1, 9, 5, 13, 3, 11, 7, 15):
                row = row0 + r
                s = (urow_v[buf, row, pl.ds(0, L)] *
                     irow_v[buf, row, pl.ds(0, L)])
                for j in range(1, EMB // L):
                    s = s + (urow_v[buf, row, pl.ds(j * L, L)] *
                             irow_v[buf, row, pl.ds(j * L, L)])
                vs.append(s)
            for d in range(4):
                vs = [combine(vs[2 * i], vs[2 * i + 1], d)
                      for i in range(len(vs) // 2)]
            acc = vs[0] + ubv[buf, pl.ds(row0, L)] + ibv[buf, pl.ds(row0, L)]
            out_v[pl.ds(ci * C + row0, L)] = jnp.maximum(acc, zero)
            return carry2

        lax.fori_loop(0, GROUPS, group_body, 0, unroll=False)
        return carry

    lax.fori_loop(0, NCHUNK, chunk_body, 0, unroll=False)

    pltpu.sync_copy(out_v, out_hbm.at[pl.ds(base, R)])


@jax.jit
def kernel(user_idx, item_idx, user_emb, item_emb, user_bias, item_bias):
    mesh = plsc.VectorSubcoreMesh(core_axis_name="c", subcore_axis_name="s")
    run = pl.kernel(
        _body,
        out_type=jax.ShapeDtypeStruct((BATCH,), jnp.float32),
        mesh=mesh,
        scratch_types=[
            pltpu.VMEM((R,), jnp.int32),
            pltpu.VMEM((R,), jnp.int32),
            pltpu.VMEM((2, C, EMB), jnp.float32),
            pltpu.VMEM((2, C, EMB), jnp.float32),
            pltpu.VMEM((2, C), jnp.float32),
            pltpu.VMEM((2, C), jnp.float32),
            pltpu.VMEM((R,), jnp.float32),
            pltpu.SemaphoreType.DMA((2, 4)),
        ],
    )
    return run(user_idx.astype(jnp.int32), item_idx.astype(jnp.int32),
               user_emb, item_emb,
               user_bias[:, 0], item_bias[:, 0])
